# baseline (speedup 1.0000x reference)
import functools

import jax
import jax.numpy as jnp
from jax import lax
from jax.experimental import pallas as pl
from jax.experimental.pallas import tpu as pltpu
from jax.experimental.pallas import tpu_sc as plsc

_N = 8192
_D = 1000
_K = 8
_INV_TEMP = 0.5
_BLOCK_ROWS = 512

_NW = 32
_RPW = _N // _NW
_CH = 32
_NCH = _RPW // _CH


def _topk_sel_kernel(cls_ref, col_ref, coef_ref):
    x = cls_ref[:]
    m = jnp.max(x, axis=-1, keepdims=True)
    ew = jnp.exp((x - m) * _INV_TEMP)
    s = jnp.sum(ew, axis=-1, keepdims=True)
    ckey = (
        _D - jax.lax.broadcasted_iota(jnp.int32, ew.shape, 1)
    ).astype(jnp.float32)
    v = jnp.ones_like(s)
    wks, coefs = [], []
    for i in range(_K):
        wk = jnp.max(jnp.where(ew >= v, ckey, 0.0), axis=-1, keepdims=True)
        wks.append(wk)
        coefs.append(v / s)
        s = s - v
        if i < _K - 1:
            ew = jnp.where(ckey == wk, 0.0, ew)
            v = jnp.max(ew, axis=-1, keepdims=True)
    col_ref[:] = (_D - jnp.concatenate(wks, axis=1)).astype(jnp.int32)
    coef_ref[:] = jnp.concatenate(coefs, axis=1)


@functools.partial(
    pl.kernel,
    out_type=jax.ShapeDtypeStruct((_N * _D,), jnp.float32),
    mesh=plsc.VectorSubcoreMesh(core_axis_name="c", subcore_axis_name="s"),
    scratch_types=[
        pltpu.VMEM((_CH * _D,), jnp.float32),
        pltpu.VMEM((_CH * _K,), jnp.int32),
        pltpu.VMEM((_CH * _K,), jnp.float32),
        pltpu.VMEM((_CH * _K,), jnp.int32),
        pltpu.VMEM((_CH * _K,), jnp.float32),
        pltpu.VMEM_SHARED((16 * _CH * _D,), jnp.float32),
    ],
)
def _scatter_sc(cols_hbm, coefs_hbm, out_hbm, buf, idxb, coefb, posb, zb,
                shared):
    sid = lax.axis_index("s")
    wid = sid * 2 + lax.axis_index("c")
    base_row = wid * _RPW
    sbase = sid * (_CH * _D)
    zero16 = jnp.zeros((16,), jnp.float32)

    def zbody(i, carry):
        buf[pl.ds(i * 16, 16)] = zero16
        return carry

    lax.fori_loop(0, _CH * _D // 16, zbody, None)
    for j in range(_CH * _K // 16):
        zb[pl.ds(j * 16, 16)] = zero16
    pltpu.sync_copy(buf, shared.at[pl.ds(sbase, _CH * _D)])

    lane = lax.iota(jnp.int32, 16)
    rowsel = lax.shift_right_logical(lane, 3) * _D
    for k in range(_NCH):
        r0 = base_row + k * _CH
        pltpu.sync_copy(cols_hbm.at[pl.ds(r0 * _K, _CH * _K)], idxb)
        pltpu.sync_copy(coefs_hbm.at[pl.ds(r0 * _K, _CH * _K)], coefb)
        for j in range(_CH * _K // 16):
            posb[pl.ds(j * 16, 16)] = (
                idxb[pl.ds(j * 16, 16)] + rowsel + (2 * j) * _D + sbase
            )
        pltpu.sync_copy(coefb, shared.at[posb])
        pltpu.sync_copy(
            shared.at[pl.ds(sbase, _CH * _D)],
            out_hbm.at[pl.ds(r0 * _D, _CH * _D)],
        )
        pltpu.sync_copy(zb, shared.at[posb])


def kernel(normu, cls):
    cols, coefs = pl.pallas_call(
        _topk_sel_kernel,
        grid=(_N // _BLOCK_ROWS,),
        in_specs=[pl.BlockSpec((_BLOCK_ROWS, _D), lambda i: (i, 0))],
        out_specs=[
            pl.BlockSpec((_BLOCK_ROWS, _K), lambda i: (i, 0)),
            pl.BlockSpec((_BLOCK_ROWS, _K), lambda i: (i, 0)),
        ],
        out_shape=[
            jax.ShapeDtypeStruct((_N, _K), jnp.int32),
            jax.ShapeDtypeStruct((_N, _K), jnp.float32),
        ],
    )(cls)
    classes = _scatter_sc(cols.reshape(-1), coefs.reshape(-1))
    return (normu, classes.reshape(_N, _D))


# speedup vs baseline: 1.6347x; 1.6347x over previous
import jax
import jax.numpy as jnp
from jax.experimental import pallas as pl

_N = 8192
_D = 1000
_K = 8
_INV_TEMP = 0.5
_BLOCK_ROWS = 512


def _topk_mask_kernel(cls_ref, out_ref):
    x = cls_ref[:]
    m = jnp.max(x, axis=-1, keepdims=True)
    ew = jnp.exp((x - m) * _INV_TEMP)
    s = jnp.sum(ew, axis=-1, keepdims=True)
    ckey = (
        _D - jax.lax.broadcasted_iota(jnp.int32, ew.shape, 1)
    ).astype(jnp.float32)
    v = jnp.ones_like(s)
    for i in range(_K):
        wk = jnp.max(jnp.where(ew >= v, ckey, 0.0), axis=-1, keepdims=True)
        ew = jnp.where(ckey == wk, -v / s, ew)
        s = s - v
        if i < _K - 1:
            v = jnp.max(ew, axis=-1, keepdims=True)
    out_ref[:] = -jnp.minimum(ew, 0.0)


def kernel(normu, cls):
    classes = pl.pallas_call(
        _topk_mask_kernel,
        grid=(_N // _BLOCK_ROWS,),
        in_specs=[pl.BlockSpec((_BLOCK_ROWS, _D), lambda i: (i, 0))],
        out_specs=pl.BlockSpec((_BLOCK_ROWS, _D), lambda i: (i, 0)),
        out_shape=jax.ShapeDtypeStruct((_N, _D), jnp.float32),
    )(cls)
    return (normu, classes)
